# baseline (speedup 1.0000x reference)
import functools

import jax
import jax.numpy as jnp
from jax.experimental import pallas as pl


def _prologue_body(adjt_ref, ws1_ref, wn1_ref, ws2_ref, wn2_ref, b1_ref,
                   b2_ref, acat_ref, ind_ref, bc_ref, bn_ref, mcat_ref):
    adjt = adjt_ref[...]
    maskT = (adjt != 0.0).astype(jnp.float32)
    deg = jnp.sum(maskT, axis=1, keepdims=True)
    inv = jnp.where(deg > 0.0, 1.0 / jnp.maximum(deg, 1.0), 0.0)
    a = inv * maskT
    a2 = jnp.dot(a, a, preferred_element_type=jnp.float32)
    acat_ref[:, 0:512] = a
    acat_ref[:, 512:1024] = a2
    ind_ref[...] = (deg > 0.0).astype(jnp.float32)
    ws1 = ws1_ref[...]
    wn1 = wn1_ref[...]
    ws2 = ws2_ref[...]
    wn2 = wn2_ref[...]
    m1 = jnp.dot(ws2, ws1, preferred_element_type=jnp.float32)
    m2 = (jnp.dot(ws2, wn1, preferred_element_type=jnp.float32)
          + jnp.dot(wn2, ws1, preferred_element_type=jnp.float32))
    m3 = jnp.dot(wn2, wn1, preferred_element_type=jnp.float32)
    b1 = b1_ref[...]
    b2 = b2_ref[...]
    bc_ref[...] = jnp.dot(b1, ws2.T, preferred_element_type=jnp.float32) + b2
    bn_ref[...] = jnp.dot(b1, wn2.T, preferred_element_type=jnp.float32)
    mcat_ref[:, 0:24] = m1.T
    mcat_ref[:, 24:48] = m2.T
    mcat_ref[:, 48:72] = m3.T


def _smallmm_body(xf_ref, mcat_ref, out_ref):
    out_ref[...] = jnp.dot(xf_ref[...], mcat_ref[...],
                           preferred_element_type=jnp.float32)


def _main_body(x1_ref, xcat_ref, acat_ref, adjt_ref, b_ref, o0_ref, z_ref):
    res = (x1_ref[...]
           + jnp.dot(acat_ref[...], xcat_ref[...],
                     preferred_element_type=jnp.float32)
           + b_ref[...])
    o0_ref[...] = 3.5 * res
    z_ref[...] = jnp.dot(adjt_ref[...], res,
                         preferred_element_type=jnp.float32)


@jax.jit
def kernel(x, adj, W_self1, W_neigh1, b1, W_self2, W_neigh2, b2):
    B, C, N, L = x.shape
    F = C * B * L
    TF = 1536
    adjT = adj.T

    acat, ind, bc, bn, mcat = pl.pallas_call(
        _prologue_body,
        out_shape=(
            jax.ShapeDtypeStruct((N, 2 * N), jnp.float32),
            jax.ShapeDtypeStruct((N, 1), jnp.float32),
            jax.ShapeDtypeStruct((1, L), jnp.float32),
            jax.ShapeDtypeStruct((1, L), jnp.float32),
            jax.ShapeDtypeStruct((L, 3 * L), jnp.float32),
        ),
    )(adjT, W_self1, W_neigh1, W_self2, W_neigh2, b1[None, :], b2[None, :])

    xf = x.reshape(B * C * N, L)
    RT = 8192
    xall = pl.pallas_call(
        _smallmm_body,
        grid=(B * C * N // RT,),
        in_specs=[
            pl.BlockSpec((RT, L), lambda i: (i, 0)),
            pl.BlockSpec((L, 3 * L), lambda i: (0, 0)),
        ],
        out_specs=pl.BlockSpec((RT, 3 * L), lambda i: (i, 0)),
        out_shape=jax.ShapeDtypeStruct((B * C * N, 3 * L), jnp.float32),
    )(xf, mcat)

    xall = xall.reshape(B, C, N, 3, L)
    x1 = xall[:, :, :, 0, :].transpose(2, 1, 0, 3).reshape(N, F)
    x2 = xall[:, :, :, 1, :].transpose(2, 1, 0, 3).reshape(N, F)
    x3 = xall[:, :, :, 2, :].transpose(2, 1, 0, 3).reshape(N, F)
    xcat = jnp.concatenate([x2, x3], axis=0)

    bmat = (jnp.tile(bc, (1, TF // L))
            + ind * jnp.tile(bn, (1, TF // L)))

    o0, z = pl.pallas_call(
        _main_body,
        grid=(F // TF,),
        in_specs=[
            pl.BlockSpec((N, TF), lambda i: (0, i)),
            pl.BlockSpec((2 * N, TF), lambda i: (0, i)),
            pl.BlockSpec((N, 2 * N), lambda i: (0, 0)),
            pl.BlockSpec((N, N), lambda i: (0, 0)),
            pl.BlockSpec((N, TF), lambda i: (0, 0)),
        ],
        out_specs=(
            pl.BlockSpec((N, TF), lambda i: (0, i)),
            pl.BlockSpec((N, TF), lambda i: (0, i)),
        ),
        out_shape=(
            jax.ShapeDtypeStruct((N, F), jnp.float32),
            jax.ShapeDtypeStruct((N, F), jnp.float32),
        ),
    )(x1, xcat, acat, adjT, bmat)

    o0r = o0.reshape(N, C, B, L).transpose(2, 1, 0, 3)
    zr = z.reshape(N, C, B, L).transpose(2, 1, 0, 3)
    out = jnp.stack([o0r, zr], axis=2)
    return out.reshape(B, C * 2, N, L)


# speedup vs baseline: 1.7200x; 1.7200x over previous
import jax
import jax.numpy as jnp
from jax.experimental import pallas as pl

_N = 512
_L = 24
_B = 64
_C = 24
_TB = 16


def _prologue_body(adj_ref, ws1_ref, wn1_ref, ws2_ref, wn2_ref, b1c_ref,
                   b2c_ref, at_ref, a2t_ref, bv_ref, mcat_ref):
    adj = adj_ref[...]
    mask = (adj != 0.0).astype(jnp.float32)
    deg = jnp.sum(mask, axis=0, keepdims=True)
    inv = jnp.where(deg > 0.0, 1.0 / jnp.maximum(deg, 1.0), 0.0)
    at = mask * inv
    at_ref[...] = at
    a2t_ref[...] = jnp.dot(at, at, preferred_element_type=jnp.float32)
    ws1 = ws1_ref[...]
    wn1 = wn1_ref[...]
    ws2 = ws2_ref[...]
    wn2 = wn2_ref[...]
    mcat_ref[0:24, :] = jnp.dot(ws2, ws1, preferred_element_type=jnp.float32)
    mcat_ref[24:48, :] = (jnp.dot(ws2, wn1, preferred_element_type=jnp.float32)
                          + jnp.dot(wn2, ws1, preferred_element_type=jnp.float32))
    mcat_ref[48:72, :] = jnp.dot(wn2, wn1, preferred_element_type=jnp.float32)
    b1c = b1c_ref[...]
    bconst = jnp.dot(ws2, b1c, preferred_element_type=jnp.float32) + b2c_ref[...]
    bneigh = jnp.dot(wn2, b1c, preferred_element_type=jnp.float32)
    ind = (deg > 0.0).astype(jnp.float32)
    bv_ref[...] = (jnp.tile(bconst, (_B, 1))
                   + jnp.tile(bneigh, (_B, 1)) * ind)


def _main_body(x_ref, at_ref, a2t_ref, adj_ref, bv_ref, mcat_ref, out_ref):
    xt = x_ref[...].reshape(_TB, _N, _L)
    xv = jnp.transpose(xt, (0, 2, 1))
    mcat_b = jnp.broadcast_to(mcat_ref[...][None], (_TB, 3 * _L, _L))
    xall = jax.lax.dot_general(
        mcat_b, xv, (((2,), (1,)), ((0,), (0,))),
        preferred_element_type=jnp.float32)
    x1 = xall[:, 0:_L, :].reshape(_TB * _L, _N)
    x2 = xall[:, _L:2 * _L, :].reshape(_TB * _L, _N)
    x3 = xall[:, 2 * _L:3 * _L, :].reshape(_TB * _L, _N)
    res = (x1
           + jnp.dot(x2, at_ref[...], preferred_element_type=jnp.float32)
           + jnp.dot(x3, a2t_ref[...], preferred_element_type=jnp.float32)
           + bv_ref[...])
    z = jnp.dot(res, adj_ref[...], preferred_element_type=jnp.float32)
    o0 = jnp.transpose((3.5 * res).reshape(_TB, _L, _N), (0, 2, 1))
    o1 = jnp.transpose(z.reshape(_TB, _L, _N), (0, 2, 1))
    out_ref[:, 0, :, :] = o0
    out_ref[:, 1, :, :] = o1


@jax.jit
def kernel(x, adj, W_self1, W_neigh1, b1, W_self2, W_neigh2, b2):
    B, C, N, L = x.shape

    at, a2t, bv, mcat = pl.pallas_call(
        _prologue_body,
        out_shape=(
            jax.ShapeDtypeStruct((N, N), jnp.float32),
            jax.ShapeDtypeStruct((N, N), jnp.float32),
            jax.ShapeDtypeStruct((B * L, N), jnp.float32),
            jax.ShapeDtypeStruct((3 * L, L), jnp.float32),
        ),
    )(adj, W_self1, W_neigh1, W_self2, W_neigh2, b1[:, None], b2[:, None])

    out = pl.pallas_call(
        _main_body,
        grid=(C, B // _TB),
        in_specs=[
            pl.BlockSpec((_TB, 1, N, L), lambda c, b: (b, c, 0, 0)),
            pl.BlockSpec((N, N), lambda c, b: (0, 0)),
            pl.BlockSpec((N, N), lambda c, b: (0, 0)),
            pl.BlockSpec((N, N), lambda c, b: (0, 0)),
            pl.BlockSpec((_TB * L, N), lambda c, b: (b, 0)),
            pl.BlockSpec((3 * L, L), lambda c, b: (0, 0)),
        ],
        out_specs=pl.BlockSpec((_TB, 2, N, L), lambda c, b: (b, c, 0, 0)),
        out_shape=jax.ShapeDtypeStruct((B, 2 * C, N, L), jnp.float32),
    )(x, at, a2t, adj, bv, mcat)

    return out


# speedup vs baseline: 11.1306x; 6.4714x over previous
import jax
import jax.numpy as jnp
from jax.experimental import pallas as pl

_N = 512
_L = 24
_B = 64
_C = 24
_TB = 16


def _prologue_body(adj_ref, ws1_ref, wn1_ref, ws2_ref, wn2_ref, b1c_ref,
                   b2c_ref, at_ref, a2t_ref, bv_ref, mcat_ref):
    adj = adj_ref[...]
    mask = (adj != 0.0).astype(jnp.float32)
    deg = jnp.sum(mask, axis=0, keepdims=True)
    inv = jnp.where(deg > 0.0, 1.0 / jnp.maximum(deg, 1.0), 0.0)
    at = mask * inv
    at_ref[...] = at
    a2t_ref[...] = jnp.dot(at, at, preferred_element_type=jnp.float32)
    ws1 = ws1_ref[...]
    wn1 = wn1_ref[...]
    ws2 = ws2_ref[...]
    wn2 = wn2_ref[...]
    mcat_ref[0:24, :] = jnp.dot(ws2, ws1, preferred_element_type=jnp.float32)
    mcat_ref[24:48, :] = (jnp.dot(ws2, wn1, preferred_element_type=jnp.float32)
                          + jnp.dot(wn2, ws1, preferred_element_type=jnp.float32))
    mcat_ref[48:72, :] = jnp.dot(wn2, wn1, preferred_element_type=jnp.float32)
    b1c = b1c_ref[...]
    bconst = jnp.dot(ws2, b1c, preferred_element_type=jnp.float32) + b2c_ref[...]
    bneigh = jnp.dot(wn2, b1c, preferred_element_type=jnp.float32)
    ind = (deg > 0.0).astype(jnp.float32)
    bv_ref[...] = (jnp.tile(bconst, (_B, 1))
                   + jnp.tile(bneigh, (_B, 1)) * ind)


def _main_body(x_ref, at_ref, a2t_ref, adj_ref, bv_ref, mcat_ref, out_ref):
    xv = x_ref[...].reshape(_TB, _L, _N)
    mcat_b = jnp.broadcast_to(mcat_ref[...][None], (_TB, 3 * _L, _L))
    xall = jax.lax.dot_general(
        mcat_b, xv, (((2,), (1,)), ((0,), (0,))),
        preferred_element_type=jnp.float32)
    x1 = xall[:, 0:_L, :].reshape(_TB * _L, _N)
    x2 = xall[:, _L:2 * _L, :].reshape(_TB * _L, _N)
    x3 = xall[:, 2 * _L:3 * _L, :].reshape(_TB * _L, _N)
    res = (x1
           + jnp.dot(x2, at_ref[...], preferred_element_type=jnp.float32)
           + jnp.dot(x3, a2t_ref[...], preferred_element_type=jnp.float32)
           + bv_ref[...])
    z = jnp.dot(res, adj_ref[...], preferred_element_type=jnp.float32)
    out_ref[:, 0, :, :] = (3.5 * res).reshape(_TB, _L, _N)
    out_ref[:, 1, :, :] = z.reshape(_TB, _L, _N)


@jax.jit
def kernel(x, adj, W_self1, W_neigh1, b1, W_self2, W_neigh2, b2):
    B, C, N, L = x.shape

    at, a2t, bv, mcat = pl.pallas_call(
        _prologue_body,
        out_shape=(
            jax.ShapeDtypeStruct((N, N), jnp.float32),
            jax.ShapeDtypeStruct((N, N), jnp.float32),
            jax.ShapeDtypeStruct((B * L, N), jnp.float32),
            jax.ShapeDtypeStruct((3 * L, L), jnp.float32),
        ),
    )(adj, W_self1, W_neigh1, W_self2, W_neigh2, b1[:, None], b2[:, None])

    xt = jnp.swapaxes(x, 2, 3)

    outt = pl.pallas_call(
        _main_body,
        grid=(C, B // _TB),
        in_specs=[
            pl.BlockSpec((_TB, 1, L, N), lambda c, b: (b, c, 0, 0)),
            pl.BlockSpec((N, N), lambda c, b: (0, 0)),
            pl.BlockSpec((N, N), lambda c, b: (0, 0)),
            pl.BlockSpec((N, N), lambda c, b: (0, 0)),
            pl.BlockSpec((_TB * L, N), lambda c, b: (b, 0)),
            pl.BlockSpec((3 * L, L), lambda c, b: (0, 0)),
        ],
        out_specs=pl.BlockSpec((_TB, 2, L, N), lambda c, b: (b, c, 0, 0)),
        out_shape=jax.ShapeDtypeStruct((B, 2 * C, L, N), jnp.float32),
    )(xt, at, a2t, adj, bv, mcat)

    return jnp.swapaxes(outt, 2, 3)


# speedup vs baseline: 16.9388x; 1.5218x over previous
import jax
import jax.numpy as jnp
from jax.experimental import pallas as pl

_N = 512
_L = 24
_B = 64
_C = 24
_TB = 64


def _prologue_body(adj_ref, ws1_ref, wn1_ref, ws2_ref, wn2_ref, b1c_ref,
                   b2c_ref, at_ref, a2t_ref, bv_ref, mcat_ref):
    adj = adj_ref[...]
    mask = (adj != 0.0).astype(jnp.float32)
    deg = jnp.sum(mask, axis=0, keepdims=True)
    inv = jnp.where(deg > 0.0, 1.0 / jnp.maximum(deg, 1.0), 0.0)
    at = mask * inv
    at_ref[...] = at
    a2t_ref[...] = jnp.dot(at, at, preferred_element_type=jnp.float32)
    ws1 = ws1_ref[...]
    wn1 = wn1_ref[...]
    ws2 = ws2_ref[...]
    wn2 = wn2_ref[...]
    mcat_ref[0:24, :] = jnp.dot(ws2, ws1, preferred_element_type=jnp.float32)
    mcat_ref[24:48, :] = (jnp.dot(ws2, wn1, preferred_element_type=jnp.float32)
                          + jnp.dot(wn2, ws1, preferred_element_type=jnp.float32))
    mcat_ref[48:72, :] = jnp.dot(wn2, wn1, preferred_element_type=jnp.float32)
    b1c = b1c_ref[...]
    bconst = jnp.dot(ws2, b1c, preferred_element_type=jnp.float32) + b2c_ref[...]
    bneigh = jnp.dot(wn2, b1c, preferred_element_type=jnp.float32)
    ind = (deg > 0.0).astype(jnp.float32)
    bv_ref[...] = (jnp.tile(bconst, (_B, 1))
                   + jnp.tile(bneigh, (_B, 1)) * ind)


def _main_body(x_ref, at_ref, a2t_ref, adj_ref, bv_ref, mcat_ref, out_ref):
    xv = x_ref[...].reshape(_TB, _L, _N)
    mcat_b = jnp.broadcast_to(mcat_ref[...][None], (_TB, 3 * _L, _L))
    xall = jax.lax.dot_general(
        mcat_b, xv, (((2,), (1,)), ((0,), (0,))),
        preferred_element_type=jnp.float32)
    x1 = xall[:, 0:_L, :].reshape(_TB * _L, _N)
    x2 = xall[:, _L:2 * _L, :].reshape(_TB * _L, _N)
    x3 = xall[:, 2 * _L:3 * _L, :].reshape(_TB * _L, _N)
    res = (x1
           + jnp.dot(x2, at_ref[...], preferred_element_type=jnp.float32)
           + jnp.dot(x3, a2t_ref[...], preferred_element_type=jnp.float32)
           + bv_ref[...])
    z = jnp.dot(res, adj_ref[...], preferred_element_type=jnp.float32)
    out_ref[:, 0, :, :] = (3.5 * res).reshape(_TB, _L, _N)
    out_ref[:, 1, :, :] = z.reshape(_TB, _L, _N)


@jax.jit
def kernel(x, adj, W_self1, W_neigh1, b1, W_self2, W_neigh2, b2):
    B, C, N, L = x.shape

    at, a2t, bv, mcat = pl.pallas_call(
        _prologue_body,
        out_shape=(
            jax.ShapeDtypeStruct((N, N), jnp.float32),
            jax.ShapeDtypeStruct((N, N), jnp.float32),
            jax.ShapeDtypeStruct((B * L, N), jnp.float32),
            jax.ShapeDtypeStruct((3 * L, L), jnp.float32),
        ),
    )(adj, W_self1, W_neigh1, W_self2, W_neigh2, b1[:, None], b2[:, None])

    xt = jnp.swapaxes(x, 2, 3)

    outt = pl.pallas_call(
        _main_body,
        grid=(C, B // _TB),
        in_specs=[
            pl.BlockSpec((_TB, 1, L, N), lambda c, b: (b, c, 0, 0)),
            pl.BlockSpec((N, N), lambda c, b: (0, 0)),
            pl.BlockSpec((N, N), lambda c, b: (0, 0)),
            pl.BlockSpec((N, N), lambda c, b: (0, 0)),
            pl.BlockSpec((_TB * L, N), lambda c, b: (b, 0)),
            pl.BlockSpec((3 * L, L), lambda c, b: (0, 0)),
        ],
        out_specs=pl.BlockSpec((_TB, 2, L, N), lambda c, b: (b, c, 0, 0)),
        out_shape=jax.ShapeDtypeStruct((B, 2 * C, L, N), jnp.float32),
    )(xt, at, a2t, adj, bv, mcat)

    return jnp.swapaxes(outt, 2, 3)
